# baseline (speedup 1.0000x reference)
import jax
import jax.numpy as jnp
from jax import lax
from jax.experimental import pallas as pl
from jax.experimental.pallas import tpu as pltpu
from jax.experimental.pallas import tpu_sc as plsc

_TL_UNKNOWN = 5
_CROSSWALK = 10
_NUM_TYPES = 13
_D = 64

_B = 4096
_TOTAL = 194
_BDRY_START = 66
_CPT = 6
_W = 512
_SWAP_A, _SWAP_B = 67, 68

_NC, _NS, _L = 2, 16, 16
_NW = _NC * _NS
_BPT = _B // _NW


def _take(v, i):
    return lax.gather(
        v, i[:, None],
        dimension_numbers=lax.GatherDimensionNumbers(
            offset_dims=(), collapsed_slice_dims=(0,), start_index_map=(0,)),
        slice_sizes=(1,),
        mode=lax.GatherScatterMode.PROMISE_IN_BOUNDS)


def _body(pack_hbm, out_hbm, tl8_v, tmpl_v, vbuf_v, tab2_v,
          tsem, osem):
    wid = lax.axis_index("s") * _NC + lax.axis_index("c")
    b0 = wid * _BPT

    base = 2 + _CPT * wid
    cols, rows = [], []
    for j in range(_CPT):
        bj = base + j
        c = jnp.where(bj == _SWAP_A, _SWAP_B,
                      jnp.where(bj == _SWAP_B, _SWAP_A, bj))
        cols.append(c)
        rows.append(jnp.where(c < _BDRY_START, 0, 1 + ((c - _BDRY_START) & 1)))
    row_lo, row_hi = rows[0], rows[0]
    for r in rows[1:]:
        row_lo = jnp.minimum(row_lo, r)
        row_hi = jnp.maximum(row_hi, r)

    tcp0 = pltpu.async_copy(
        pack_hbm.at[pl.ds(row_lo * _D, _D)], tmpl_v.at[0], tsem)
    tcp1 = pltpu.async_copy(
        pack_hbm.at[pl.ds(row_hi * _D, _D)], tmpl_v.at[1], tsem)
    pltpu.sync_copy(pack_hbm.at[pl.ds(3 * _D + 8, 8)], tl8_v)
    pltpu.sync_copy(pack_hbm.at[pl.ds(3 * _D, 8)], tab2_v)
    trow, tcol = wid // 4, (wid % 4) * _BPT

    idx = []
    for k in range(_BPT // _L):
        t16 = tl8_v[trow, pl.ds(tcol + k * _L, _L)]
        idx.append(jnp.clip(t16.astype(jnp.int32) + _TL_UNKNOWN,
                            0, _NUM_TYPES - 1))

    for d in range(_D):
        col_d = tab2_v[d // 32, pl.ds((d % 32) * _L, _L)]
        for k in range(_BPT // _L):
            vbuf_v[d, pl.ds(k * _L, _L)] = _take(col_d, idx[k])

    cps = [
        pltpu.async_copy(vbuf_v, out_hbm.at[0, :, pl.ds(b0, _BPT)], osem),
        pltpu.async_copy(vbuf_v, out_hbm.at[1, :, pl.ds(b0, _BPT)], osem),
    ]
    tcp0.wait()
    tcp1.wait()
    for j in range(_CPT):
        rsel = jnp.where(rows[j] == row_hi, 1, 0)
        for k in range(_B // _W):
            cps.append(pltpu.async_copy(
                tmpl_v.at[rsel], out_hbm.at[cols[j], :, pl.ds(k * _W, _W)],
                osem))
    for cp in cps:
        cp.wait()


@jax.jit
def _emb_fill(pack):
    fn = pl.kernel(
        _body,
        out_type=jax.ShapeDtypeStruct((_TOTAL, _D, _B), jnp.float32),
        mesh=plsc.VectorSubcoreMesh(core_axis_name="c", subcore_axis_name="s"),
        scratch_types=[
            pltpu.VMEM((8, _W), jnp.float32),
            pltpu.VMEM((2, _D, _W), jnp.float32),
            pltpu.VMEM((_D, _BPT), jnp.float32),
            pltpu.VMEM((8, _W), jnp.float32),
            pltpu.SemaphoreType.DMA,
            pltpu.SemaphoreType.DMA,
        ],
    )
    outT = fn(pack)
    return jnp.transpose(outT, (2, 0, 1))


def kernel(type, lanes_mid, crosswalks, lanes, emb_table):
    del type, crosswalks, lanes
    tl = lanes_mid[:, 0, 0, -1]
    tabT = jnp.pad(emb_table.T, ((0, 0), (0, _L - _NUM_TYPES)))
    pack = jnp.concatenate([
        jnp.broadcast_to(
            emb_table[_CROSSWALK:_CROSSWALK + 3, :, None],
            (3, _D, _W)).reshape(3 * _D, _W),
        tabT.reshape(2, _W),
        jnp.zeros((6, _W), jnp.float32),
        tl.reshape(8, _W),
    ], axis=0)
    return _emb_fill(pack)


# speedup vs baseline: 1.0098x; 1.0098x over previous
import jax
import jax.numpy as jnp
from jax import lax
from jax.experimental import pallas as pl
from jax.experimental.pallas import tpu as pltpu
from jax.experimental.pallas import tpu_sc as plsc

_TL_UNKNOWN = 5
_CROSSWALK = 10
_NUM_TYPES = 13
_D = 64

_B = 4096
_TOTAL = 194
_BDRY_START = 66
_CPT = 6
_W = 512
_SWAP_A, _SWAP_B = 67, 68

_NC, _NS, _L = 2, 16, 16
_NW = _NC * _NS
_BPT = _B // _NW


def _take(v, i):
    return lax.gather(
        v, i[:, None],
        dimension_numbers=lax.GatherDimensionNumbers(
            offset_dims=(), collapsed_slice_dims=(0,), start_index_map=(0,)),
        slice_sizes=(1,),
        mode=lax.GatherScatterMode.PROMISE_IN_BOUNDS)


def _body(tl_hbm, pack_hbm, out_hbm, tl_v, tmpl_v, vbuf_v, tab2_v,
          tsem, osem):
    wid = lax.axis_index("s") * _NC + lax.axis_index("c")
    b0 = wid * _BPT

    base = 2 + _CPT * wid
    cols, rows = [], []
    for j in range(_CPT):
        bj = base + j
        c = jnp.where(bj == _SWAP_A, _SWAP_B,
                      jnp.where(bj == _SWAP_B, _SWAP_A, bj))
        cols.append(c)
        rows.append(jnp.where(c < _BDRY_START, 0, 1 + ((c - _BDRY_START) & 1)))
    row_lo, row_hi = rows[0], rows[0]
    for r in rows[1:]:
        row_lo = jnp.minimum(row_lo, r)
        row_hi = jnp.maximum(row_hi, r)

    tcp0 = pltpu.async_copy(
        pack_hbm.at[pl.ds(row_lo * _D, _D)], tmpl_v.at[0], tsem)
    tcp1 = pltpu.async_copy(
        pack_hbm.at[pl.ds(row_hi * _D, _D)], tmpl_v.at[1], tsem)
    pltpu.sync_copy(tl_hbm.at[pl.ds(b0, _BPT)], tl_v)
    pltpu.sync_copy(pack_hbm.at[pl.ds(3 * _D, 8)], tab2_v)

    idx = []
    for k in range(_BPT // _L):
        t16 = tl_v[pl.ds(k * _L, _L)]
        idx.append(jnp.clip(t16.astype(jnp.int32) + _TL_UNKNOWN,
                            0, _NUM_TYPES - 1))

    for d in range(_D):
        col_d = tab2_v[d // 32, pl.ds((d % 32) * _L, _L)]
        for k in range(_BPT // _L):
            vbuf_v[d, pl.ds(k * _L, _L)] = _take(col_d, idx[k])

    cps = [
        pltpu.async_copy(vbuf_v, out_hbm.at[0, :, pl.ds(b0, _BPT)], osem),
        pltpu.async_copy(vbuf_v, out_hbm.at[1, :, pl.ds(b0, _BPT)], osem),
    ]
    tcp0.wait()
    tcp1.wait()
    for j in range(_CPT):
        rsel = jnp.where(rows[j] == row_hi, 1, 0)
        for k in range(_B // _W):
            cps.append(pltpu.async_copy(
                tmpl_v.at[rsel], out_hbm.at[cols[j], :, pl.ds(k * _W, _W)],
                osem))
    for cp in cps:
        cp.wait()


@jax.jit
def _emb_fill(tl, pack):
    fn = pl.kernel(
        _body,
        out_type=jax.ShapeDtypeStruct((_TOTAL, _D, _B), jnp.float32),
        mesh=plsc.VectorSubcoreMesh(core_axis_name="c", subcore_axis_name="s"),
        scratch_types=[
            pltpu.VMEM((_BPT,), jnp.float32),
            pltpu.VMEM((2, _D, _W), jnp.float32),
            pltpu.VMEM((_D, _BPT), jnp.float32),
            pltpu.VMEM((8, _W), jnp.float32),
            pltpu.SemaphoreType.DMA,
            pltpu.SemaphoreType.DMA,
        ],
    )
    outT = fn(tl, pack)
    return jnp.transpose(outT, (2, 0, 1))


def kernel(type, lanes_mid, crosswalks, lanes, emb_table):
    del type, crosswalks, lanes
    tl = lanes_mid[:, 0, 0, -1]
    tabT = jnp.pad(emb_table.T, ((0, 0), (0, _L - _NUM_TYPES)))
    pack = jnp.concatenate([
        jnp.broadcast_to(
            emb_table[_CROSSWALK:_CROSSWALK + 3, :, None],
            (3, _D, _W)).reshape(3 * _D, _W),
        tabT.reshape(2, _W),
        jnp.zeros((6, _W), jnp.float32),
    ], axis=0)
    return _emb_fill(tl, pack)
